# baseline (speedup 1.0000x reference)
import jax
import jax.numpy as jnp
from jax import lax
from jax.experimental import pallas as pl
from jax.experimental.pallas import tpu as pltpu
from jax.experimental.pallas import tpu_sc as plsc

B = 16
H_KV = 2
MAX_LEN = 2048
HEAD_DIM = 128
NW = 32
ROWS = 4 * B * H_KV * MAX_LEN


CHUNK = 128
NBUF = 3
NCHUNK = 4 * MAX_LEN // CHUNK


class _Ring:
    def __init__(self, bufs, sem_in, sem_out, chunk_ids, src_slice, dst_slice):
        self.bufs = bufs
        self.sem_in = sem_in
        self.sem_out = sem_out
        self.ids = chunk_ids
        self.src = src_slice
        self.dst = dst_slice
        self.n = len(chunk_ids)
        self.nbuf = len(bufs)
        self.in_cp = [None] * self.nbuf
        self.out_cp = [None] * self.nbuf

    def prime(self):
        for j in range(min(self.nbuf, self.n)):
            self.in_cp[j] = pltpu.async_copy(
                self.src(self.ids[j]), self.bufs[j], self.sem_in.at[j])

    def step(self, i):
        if i >= self.n:
            return
        j = i % self.nbuf
        self.in_cp[j].wait()
        self.out_cp[j] = pltpu.async_copy(
            self.bufs[j], self.dst(self.ids[i]), self.sem_out.at[j])
        nxt = i + self.nbuf
        if nxt < self.n:
            self.out_cp[j].wait()
            self.in_cp[j] = pltpu.async_copy(
                self.src(self.ids[nxt]), self.bufs[j], self.sem_in.at[j])

    def drain(self):
        for i in range(max(0, self.n - self.nbuf), self.n):
            self.out_cp[i % self.nbuf].wait()


def _body(c0, c1, c2, c3, rows_hbm, idx_hbm, out,
          tbufs0, tbufs1, tbufs2, shared, idx_v, rows_v,
          sem_in_a, sem_out_a, sem_in_b, sem_out_b, sem_row):
    s = lax.axis_index("s")
    w = s * 2 + lax.axis_index("c")
    caches = (c0, c1, c2, c3)

    def src_slice(i):
        c, k = divmod(i, MAX_LEN // CHUNK)
        return caches[c].at[pl.ds(w * MAX_LEN + k * CHUNK, CHUNK)]

    def dst_slice(i):
        c, k = divmod(i, MAX_LEN // CHUNK)
        return out.at[pl.ds((c * NW + w) * MAX_LEN + k * CHUNK, CHUNK)]

    ring_a = _Ring((tbufs0, tbufs1, tbufs2), sem_in_a, sem_out_a,
                   [i for i in range(NCHUNK) if i % 2 == 0],
                   src_slice, dst_slice)
    ring_b = _Ring(tuple(shared.at[s, j] for j in range(NBUF)),
                   sem_in_b, sem_out_b,
                   [i for i in range(NCHUNK) if i % 2 == 1],
                   src_slice, dst_slice)
    ring_a.prime()
    ring_b.prime()
    for i in range(NCHUNK // 2):
        ring_a.step(i)
        ring_b.step(i)
    ring_a.drain()
    ring_b.drain()
    pltpu.sync_copy(idx_hbm.at[w], idx_v)
    pltpu.sync_copy(rows_hbm.at[w], rows_v)
    pltpu.async_copy(rows_v, out.at[idx_v], sem_row).wait()


def kernel(k_cache_0, v_cache_0, k_cache_1, v_cache_1,
           new_k_0, new_v_0, new_k_1, new_v_1,
           position_ids, seq_ids):
    del seq_ids
    pos = position_ids[:, 0].astype(jnp.int32)

    flat = lambda c: c.reshape(B * H_KV * MAX_LEN, HEAD_DIM)
    new_rows = jnp.stack(
        [new_k_0[:, :, 0], new_v_0[:, :, 0], new_k_1[:, :, 0], new_v_1[:, :, 0]],
        axis=2,
    ).reshape(B * H_KV, 4, HEAD_DIM)
    bh = jnp.arange(B * H_KV, dtype=jnp.int32)
    c = jnp.arange(4, dtype=jnp.int32)
    dest_idx = (c[None, :] * NW + bh[:, None]) * MAX_LEN + pos[bh // H_KV][:, None]

    mesh = plsc.VectorSubcoreMesh(core_axis_name="c", subcore_axis_name="s")
    out = pl.kernel(
        _body,
        out_type=jax.ShapeDtypeStruct((ROWS, HEAD_DIM), jnp.float32),
        mesh=mesh,
        scratch_types=[
            pltpu.VMEM((CHUNK, HEAD_DIM), jnp.float32),
            pltpu.VMEM((CHUNK, HEAD_DIM), jnp.float32),
            pltpu.VMEM((CHUNK, HEAD_DIM), jnp.float32),
            pltpu.VMEM_SHARED((16, NBUF, CHUNK, HEAD_DIM), jnp.float32),
            pltpu.VMEM((4,), jnp.int32),
            pltpu.VMEM((4, HEAD_DIM), jnp.float32),
            pltpu.SemaphoreType.DMA((NBUF,)),
            pltpu.SemaphoreType.DMA((NBUF,)),
            pltpu.SemaphoreType.DMA((NBUF,)),
            pltpu.SemaphoreType.DMA((NBUF,)),
            pltpu.SemaphoreType.DMA,
        ],
    )(flat(k_cache_0), flat(v_cache_0), flat(k_cache_1), flat(v_cache_1),
      new_rows, dest_idx)
    return out.reshape(4, B, H_KV, MAX_LEN, HEAD_DIM)


# speedup vs baseline: 1.0477x; 1.0477x over previous
import jax
import jax.numpy as jnp
from jax import lax
from jax.experimental import pallas as pl
from jax.experimental.pallas import tpu as pltpu
from jax.experimental.pallas import tpu_sc as plsc

B = 16
H_KV = 2
MAX_LEN = 2048
HEAD_DIM = 128
NW = 32
ROWS = 4 * B * H_KV * MAX_LEN


CHUNK = 256
NBUF = 3
NCHUNK = 4 * MAX_LEN // CHUNK


class _Ring:
    def __init__(self, bufs, sem_in, sem_out, chunk_ids, src_slice, dst_slice):
        self.bufs = bufs
        self.sem_in = sem_in
        self.sem_out = sem_out
        self.ids = chunk_ids
        self.src = src_slice
        self.dst = dst_slice
        self.n = len(chunk_ids)
        self.nbuf = len(bufs)
        self.in_cp = [None] * self.nbuf
        self.out_cp = [None] * self.nbuf

    def prime(self):
        for j in range(min(self.nbuf, self.n)):
            self.in_cp[j] = pltpu.async_copy(
                self.src(self.ids[j]), self.bufs[j], self.sem_in.at[j])

    def step(self, i):
        if i >= self.n:
            return
        j = i % self.nbuf
        self.in_cp[j].wait()
        self.out_cp[j] = pltpu.async_copy(
            self.bufs[j], self.dst(self.ids[i]), self.sem_out.at[j])
        nxt = i + self.nbuf
        if nxt < self.n:
            self.out_cp[j].wait()
            self.in_cp[j] = pltpu.async_copy(
                self.src(self.ids[nxt]), self.bufs[j], self.sem_in.at[j])

    def drain(self):
        for i in range(max(0, self.n - self.nbuf), self.n):
            self.out_cp[i % self.nbuf].wait()


def _body(c0, c1, c2, c3, rows_hbm, idx_hbm, out,
          shared, idx_v, rows_v, sem_in, sem_out, sem_row, sem_pre):
    s = lax.axis_index("s")
    w = s * 2 + lax.axis_index("c")
    caches = (c0, c1, c2, c3)

    def src_slice(i):
        c, k = divmod(i, MAX_LEN // CHUNK)
        return caches[c].at[pl.ds(w * MAX_LEN + k * CHUNK, CHUNK)]

    def dst_slice(i):
        c, k = divmod(i, MAX_LEN // CHUNK)
        return out.at[pl.ds((c * NW + w) * MAX_LEN + k * CHUNK, CHUNK)]

    pre_idx = pltpu.async_copy(idx_hbm.at[w], idx_v, sem_pre)
    pre_rows = pltpu.async_copy(rows_hbm.at[w], rows_v, sem_pre)

    ring = _Ring(tuple(shared.at[s, j] for j in range(NBUF)),
                 sem_in, sem_out, list(range(NCHUNK)), src_slice, dst_slice)
    ring.prime()
    for i in range(NCHUNK):
        ring.step(i)
    ring.drain()
    pre_idx.wait()
    pre_rows.wait()
    pltpu.async_copy(rows_v, out.at[idx_v], sem_row).wait()


def kernel(k_cache_0, v_cache_0, k_cache_1, v_cache_1,
           new_k_0, new_v_0, new_k_1, new_v_1,
           position_ids, seq_ids):
    del seq_ids
    pos = position_ids[:, 0].astype(jnp.int32)

    flat = lambda c: c.reshape(B * H_KV * MAX_LEN, HEAD_DIM)
    new_rows = jnp.stack(
        [new_k_0[:, :, 0], new_v_0[:, :, 0], new_k_1[:, :, 0], new_v_1[:, :, 0]],
        axis=2,
    ).reshape(B * H_KV, 4, HEAD_DIM)
    bh = jnp.arange(B * H_KV, dtype=jnp.int32)
    c = jnp.arange(4, dtype=jnp.int32)
    dest_idx = (c[None, :] * NW + bh[:, None]) * MAX_LEN + pos[bh // H_KV][:, None]

    mesh = plsc.VectorSubcoreMesh(core_axis_name="c", subcore_axis_name="s")
    out = pl.kernel(
        _body,
        out_type=jax.ShapeDtypeStruct((ROWS, HEAD_DIM), jnp.float32),
        mesh=mesh,
        scratch_types=[
            pltpu.VMEM_SHARED((16, NBUF, CHUNK, HEAD_DIM), jnp.float32),
            pltpu.VMEM((4,), jnp.int32),
            pltpu.VMEM((4, HEAD_DIM), jnp.float32),
            pltpu.SemaphoreType.DMA((NBUF,)),
            pltpu.SemaphoreType.DMA((NBUF,)),
            pltpu.SemaphoreType.DMA,
            pltpu.SemaphoreType.DMA,
        ],
    )(flat(k_cache_0), flat(v_cache_0), flat(k_cache_1), flat(v_cache_1),
      new_rows, dest_idx)
    return out.reshape(4, B, H_KV, MAX_LEN, HEAD_DIM)
